# baseline (speedup 1.0000x reference)
import jax
import jax.numpy as jnp
from jax import lax
from jax.experimental import pallas as pl
from jax.experimental.pallas import tpu as pltpu
from jax.experimental.pallas import tpu_sc as plsc

H = 384
W = 384
C = 96
HW = H * W
NW = 32
ROWS_PER_W = H // NW
CHUNK = 96
CPR = W // CHUNK
GROUPS = CHUNK // 16
NCHUNK = ROWS_PER_W * CPR


def _sc_body(table, dx_hbm, dy_hbm, out_hbm,
             dxv, dyv, idx4, w4, rowsf, outbuf,
             gsemA, gsemB, osemA, osemB):
    wid = lax.axis_index("s") * 2 + lax.axis_index("c")
    row0 = wid * ROWS_PER_W
    src0 = (H - ROWS_PER_W) - row0

    pltpu.sync_copy(dx_hbm.at[pl.ds(src0 * W, ROWS_PER_W * W)], dxv)
    pltpu.sync_copy(dy_hbm.at[pl.ds(src0 * W, ROWS_PER_W * W)], dyv)

    iota = lax.iota(jnp.int32, 16)

    def floor_i32(v):
        t = v.astype(jnp.int32)
        return jnp.where(v < t.astype(jnp.float32), t - 1, t)

    def compute_idx(n, slot):
        i = lax.shift_right_logical(n, 2)
        cb = lax.bitwise_and(n, CPR - 1)
        li = (ROWS_PER_W - 1) - i
        rp = (H - 1) - (row0 + i)
        rp_f = jnp.full((16,), rp, dtype=jnp.int32).astype(jnp.float32)
        rowoff = li * W + cb * CHUNK
        colbase0 = cb * CHUNK

        def idx_group(g, c2):
            dxg = dxv[pl.ds(rowoff + g * 16, 16)]
            dyg = dyv[pl.ds(rowoff + g * 16, 16)]
            colf = (iota + (colbase0 + g * 16)).astype(jnp.float32)
            x = (dxg + colf) + 1.0
            y = (dyg + rp_f) + 1.0
            fx = floor_i32(x)
            fy = floor_i32(y)
            px0 = jnp.clip(fx, 0, W + 1)
            px1 = jnp.clip(fx + 1, 0, W + 1)
            py0 = jnp.clip(fy, 0, H + 1)
            py1 = jnp.clip(fy + 1, 0, H + 1)
            dxw = px1.astype(jnp.float32) - x
            dyw = py1.astype(jnp.float32) - y
            wa = dxw * dyw
            wb = dxw * (1.0 - dyw)
            wc = (1.0 - dxw) * dyw
            wd = (1.0 - dxw) * (1.0 - dyw)
            okx0 = (px0 >= 1) & (px0 <= W)
            okx1 = (px1 >= 1) & (px1 <= W)
            oky0 = (py0 >= 1) & (py0 <= H)
            oky1 = (py1 >= 1) & (py1 <= H)
            cx0 = jnp.clip(px0 - 1, 0, W - 1)
            cx1 = jnp.clip(px1 - 1, 0, W - 1)
            cy0 = jnp.clip(py0 - 1, 0, H - 1) * W
            cy1 = jnp.clip(py1 - 1, 0, H - 1) * W
            sl = pl.ds(g * 16, 16)
            idx4[slot, 0, sl] = cy0 + cx0
            idx4[slot, 1, sl] = cy0 + cx1
            idx4[slot, 2, sl] = cy1 + cx0
            idx4[slot, 3, sl] = cy1 + cx1
            zero = jnp.zeros((16,), jnp.float32)
            w4[slot, 0, sl] = jnp.where(okx0 & oky0, wa, zero)
            w4[slot, 1, sl] = jnp.where(okx1 & oky0, wb, zero)
            w4[slot, 2, sl] = jnp.where(okx0 & oky1, wc, zero)
            w4[slot, 3, sl] = jnp.where(okx1 & oky1, wd, zero)
            return c2

        lax.fori_loop(0, GROUPS, idx_group, 0)

    def fire_gathers(slot, gsem):
        return [
            pltpu.async_copy(table.at[idx4.at[slot, t]], rowsf.at[slot, t],
                             gsem)
            for t in range(4)
        ]

    def wait_gathers(slot, gsem):
        for t in range(4):
            pltpu.make_async_copy(table.at[idx4.at[slot, t]],
                                  rowsf.at[slot, t], gsem).wait()

    def combine(slot):
        r32 = [rowsf.at[slot, t].bitcast(jnp.int32) for t in range(4)]
        mhi = jnp.full((16,), -65536, dtype=jnp.int32)

        def g_body(g, c2):
            sl = pl.ds(g * 16, 16)
            wa_v = w4[slot, 0, sl]
            wb_v = w4[slot, 1, sl]
            wc_v = w4[slot, 2, sl]
            wd_v = w4[slot, 3, sl]
            for p16 in range(16):
                p = g * 16 + p16
                wa_s = wa_v[p16]
                wb_s = wb_v[p16]
                wc_s = wc_v[p16]
                wd_s = wd_v[p16]
                for kw in range(C // 32):
                    slk = pl.ds(16 * kw, 16)
                    v0 = r32[0][p, slk]
                    v1 = r32[1][p, slk]
                    v2 = r32[2][p, slk]
                    v3 = r32[3][p, slk]
                    f = jnp.float32
                    e0 = lax.bitcast_convert_type(lax.shift_left(v0, 16), f)
                    e1 = lax.bitcast_convert_type(lax.shift_left(v1, 16), f)
                    e2 = lax.bitcast_convert_type(lax.shift_left(v2, 16), f)
                    e3 = lax.bitcast_convert_type(lax.shift_left(v3, 16), f)
                    o0 = lax.bitcast_convert_type(v0 & mhi, f)
                    o1 = lax.bitcast_convert_type(v1 & mhi, f)
                    o2 = lax.bitcast_convert_type(v2 & mhi, f)
                    o3 = lax.bitcast_convert_type(v3 & mhi, f)
                    acc_e = wa_s * e0 + wb_s * e1 + wc_s * e2 + wd_s * e3
                    acc_o = wa_s * o0 + wb_s * o1 + wc_s * o2 + wd_s * o3
                    outbuf[slot, p, pl.ds(32 * kw, 16)] = acc_e
                    outbuf[slot, p, pl.ds(32 * kw + 16, 16)] = acc_o
            return c2

        lax.fori_loop(0, GROUPS, g_body, 0)

    def out_slice(n):
        i = lax.shift_right_logical(n, 2)
        cb = lax.bitwise_and(n, CPR - 1)
        pixbase = (row0 + i) * W + cb * CHUNK
        return out_hbm.at[pl.ds(pixbase, CHUNK)]

    def fire_out(slot, n, osem):
        pltpu.async_copy(outbuf.at[slot], out_slice(n), osem)

    def wait_out(slot, n_prev, osem):
        pltpu.make_async_copy(outbuf.at[slot], out_slice(n_prev), osem).wait()

    compute_idx(0, 0)
    fire_gathers(0, gsemA)

    def m_body(m, carry):
        nA = 2 * m
        nB = 2 * m + 1
        compute_idx(nB, 1)
        fire_gathers(1, gsemB)
        wait_gathers(0, gsemA)

        @pl.when(m > 0)
        def _():
            wait_out(0, nA - 2, osemA)

        combine(0)
        fire_out(0, nA, osemA)

        @pl.when(m < (NCHUNK // 2 - 1))
        def _():
            compute_idx(nB + 1, 0)
            fire_gathers(0, gsemA)

        wait_gathers(1, gsemB)

        @pl.when(m > 0)
        def _():
            wait_out(1, nB - 2, osemB)

        combine(1)
        fire_out(1, nB, osemB)
        return carry

    lax.fori_loop(0, NCHUNK // 2, m_body, 0)
    wait_out(0, NCHUNK - 2, osemA)
    wait_out(1, NCHUNK - 1, osemB)


@jax.jit
def _sc_sample(table, dx, dy):
    mesh = plsc.VectorSubcoreMesh(
        core_axis_name="c", subcore_axis_name="s", num_cores=2,
        num_subcores=16)
    return pl.kernel(
        _sc_body,
        out_type=jax.ShapeDtypeStruct((HW, C), jnp.float32),
        mesh=mesh,
        compiler_params=pltpu.CompilerParams(use_tc_tiling_on_sc=False),
        scratch_types=[
            pltpu.VMEM((ROWS_PER_W * W,), jnp.float32),
            pltpu.VMEM((ROWS_PER_W * W,), jnp.float32),
            pltpu.VMEM((2, 4, CHUNK), jnp.int32),
            pltpu.VMEM((2, 4, CHUNK), jnp.float32),
            pltpu.VMEM((2, 4, CHUNK, C), jnp.bfloat16),
            pltpu.VMEM((2, CHUNK, C), jnp.float32),
            pltpu.SemaphoreType.DMA,
            pltpu.SemaphoreType.DMA,
            pltpu.SemaphoreType.DMA,
            pltpu.SemaphoreType.DMA,
        ],
    )(table, dx, dy)


def kernel(I, dx_t, dy_t):
    perm = I[0].reshape(C // 32, 2, 16, HW).transpose(0, 2, 1, 3)
    table = jnp.transpose(perm.reshape(C, HW).astype(jnp.bfloat16))
    out_flat = _sc_sample(table, dx_t.reshape(HW), dy_t.reshape(HW))
    return jnp.transpose(out_flat.reshape(1, H, W, C), (0, 3, 1, 2))


# speedup vs baseline: 1.8237x; 1.8237x over previous
import jax
import jax.numpy as jnp
from jax import lax
from jax.experimental import pallas as pl
from jax.experimental.pallas import tpu as pltpu
from jax.experimental.pallas import tpu_sc as plsc

H = 384
W = 384
C = 96
CP = 128
HW = H * W
NW = 32
ROWS_PER_W = H // NW
CHUNK = 64
CPR = W // CHUNK
GROUPS = CHUNK // 16


def _sc_body(table, dx_hbm, dy_hbm, out_hbm,
             dxv, dyv, idx4, w4, rowsf, outbuf,
             gsemA, gsemB, osemA, osemB):
    wid = lax.axis_index("s") * 2 + lax.axis_index("c")
    row0 = wid * ROWS_PER_W
    src0 = (H - ROWS_PER_W) - row0

    pltpu.sync_copy(dx_hbm.at[pl.ds(src0 * W, ROWS_PER_W * W)], dxv)
    pltpu.sync_copy(dy_hbm.at[pl.ds(src0 * W, ROWS_PER_W * W)], dyv)

    iota = lax.iota(jnp.int32, 16)

    def floor_i32(v):
        t = v.astype(jnp.int32)
        return jnp.where(v < t.astype(jnp.float32), t - 1, t)

    def compute_idx(i, cb, slot):
        li = (ROWS_PER_W - 1) - i
        rp = (H - 1) - (row0 + i)
        rp_f = jnp.full((16,), rp, dtype=jnp.int32).astype(jnp.float32)
        rowoff = li * W + cb * CHUNK
        colbase0 = cb * CHUNK

        def idx_group(g, c2):
            dxg = dxv[pl.ds(rowoff + g * 16, 16)]
            dyg = dyv[pl.ds(rowoff + g * 16, 16)]
            colf = (iota + (colbase0 + g * 16)).astype(jnp.float32)
            x = (dxg + colf) + 1.0
            y = (dyg + rp_f) + 1.0
            fx = floor_i32(x)
            fy = floor_i32(y)
            px0 = jnp.clip(fx, 0, W + 1)
            px1 = jnp.clip(fx + 1, 0, W + 1)
            py0 = jnp.clip(fy, 0, H + 1)
            py1 = jnp.clip(fy + 1, 0, H + 1)
            dxw = px1.astype(jnp.float32) - x
            dyw = py1.astype(jnp.float32) - y
            wa = dxw * dyw
            wb = dxw * (1.0 - dyw)
            wc = (1.0 - dxw) * dyw
            wd = (1.0 - dxw) * (1.0 - dyw)
            okx0 = (px0 >= 1) & (px0 <= W)
            okx1 = (px1 >= 1) & (px1 <= W)
            oky0 = (py0 >= 1) & (py0 <= H)
            oky1 = (py1 >= 1) & (py1 <= H)
            cx0 = jnp.clip(px0 - 1, 0, W - 1)
            cx1 = jnp.clip(px1 - 1, 0, W - 1)
            cy0 = jnp.clip(py0 - 1, 0, H - 1) * W
            cy1 = jnp.clip(py1 - 1, 0, H - 1) * W
            sl = pl.ds(g * 16, 16)
            idx4[slot, 0, sl] = cy0 + cx0
            idx4[slot, 1, sl] = cy0 + cx1
            idx4[slot, 2, sl] = cy1 + cx0
            idx4[slot, 3, sl] = cy1 + cx1
            zero = jnp.zeros((16,), jnp.float32)
            w4[slot, 0, sl] = jnp.where(okx0 & oky0, wa, zero)
            w4[slot, 1, sl] = jnp.where(okx1 & oky0, wb, zero)
            w4[slot, 2, sl] = jnp.where(okx0 & oky1, wc, zero)
            w4[slot, 3, sl] = jnp.where(okx1 & oky1, wd, zero)
            return c2

        lax.fori_loop(0, GROUPS, idx_group, 0)

    def fire_gathers(slot, gsem):
        for t in range(4):
            pltpu.async_copy(table.at[idx4.at[slot, t]], rowsf.at[slot, t],
                             gsem)

    def wait_gathers(slot, gsem):
        for t in range(4):
            pltpu.make_async_copy(table.at[idx4.at[slot, t]],
                                  rowsf.at[slot, t], gsem).wait()

    def combine(slot):
        def g_body(g, c2):
            sl = pl.ds(g * 16, 16)
            wa_v = w4[slot, 0, sl]
            wb_v = w4[slot, 1, sl]
            wc_v = w4[slot, 2, sl]
            wd_v = w4[slot, 3, sl]
            for p16 in range(16):
                p = g * 16 + p16
                wa_s = wa_v[p16]
                wb_s = wb_v[p16]
                wc_s = wc_v[p16]
                wd_s = wd_v[p16]
                for k in range(C // 16):
                    slk = pl.ds(k * 16, 16)
                    acc = wa_s * rowsf[slot, 0, p, slk] \
                        + wb_s * rowsf[slot, 1, p, slk] \
                        + wc_s * rowsf[slot, 2, p, slk] \
                        + wd_s * rowsf[slot, 3, p, slk]
                    outbuf[slot, p, slk] = acc
            return c2

        lax.fori_loop(0, GROUPS, g_body, 0)

    def out_slice(i, cb):
        pixbase = (row0 + i) * W + cb * CHUNK
        return out_hbm.at[pl.ds(pixbase, CHUNK)]

    def fire_out(slot, i, cb, osem):
        pltpu.async_copy(outbuf.at[slot], out_slice(i, cb), osem)

    def wait_out(slot, i, cb, osem):
        pltpu.make_async_copy(outbuf.at[slot], out_slice(i, cb), osem).wait()

    gsems = (gsemA, gsemB)
    osems = (osemA, osemB)

    compute_idx(0, 0, 0)
    fire_gathers(0, gsemA)

    def row_body(i, carry):
        for cb in range(CPR):
            slot = cb & 1
            nslot = 1 - slot
            if cb < CPR - 1:
                compute_idx(i, cb + 1, nslot)
                fire_gathers(nslot, gsems[nslot])
            else:
                @pl.when(i < ROWS_PER_W - 1)
                def _():
                    compute_idx(i + 1, 0, nslot)
                    fire_gathers(nslot, gsems[nslot])

            wait_gathers(slot, gsems[slot])

            if cb >= 2:
                wait_out(slot, i, cb - 2, osems[slot])
            else:
                @pl.when(i > 0)
                def _():
                    wait_out(slot, i - 1, cb - 2 + CPR, osems[slot])

            combine(slot)
            fire_out(slot, i, cb, osems[slot])
        return carry

    lax.fori_loop(0, ROWS_PER_W, row_body, 0)
    wait_out(0, ROWS_PER_W - 1, CPR - 2, osemA)
    wait_out(1, ROWS_PER_W - 1, CPR - 1, osemB)


@jax.jit
def _sc_sample(table, dx, dy):
    mesh = plsc.VectorSubcoreMesh(
        core_axis_name="c", subcore_axis_name="s", num_cores=2,
        num_subcores=16)
    return pl.kernel(
        _sc_body,
        out_type=jax.ShapeDtypeStruct((HW, C), jnp.float32),
        mesh=mesh,
        compiler_params=pltpu.CompilerParams(use_tc_tiling_on_sc=True),
        scratch_types=[
            pltpu.VMEM((ROWS_PER_W * W,), jnp.float32),
            pltpu.VMEM((ROWS_PER_W * W,), jnp.float32),
            pltpu.VMEM((2, 4, CHUNK), jnp.int32),
            pltpu.VMEM((2, 4, CHUNK), jnp.float32),
            pltpu.VMEM((2, 4, CHUNK, CP), jnp.float32),
            pltpu.VMEM((2, CHUNK, C), jnp.float32),
            pltpu.SemaphoreType.DMA,
            pltpu.SemaphoreType.DMA,
            pltpu.SemaphoreType.DMA,
            pltpu.SemaphoreType.DMA,
        ],
    )(table, dx, dy)


def kernel(I, dx_t, dy_t):
    t = jnp.transpose(I[0].reshape(C, HW))
    table = jnp.pad(t, ((0, 0), (0, CP - C)))
    out_flat = _sc_sample(table, dx_t.reshape(HW), dy_t.reshape(HW))
    return jnp.transpose(out_flat.reshape(1, H, W, C), (0, 3, 1, 2))


# speedup vs baseline: 1.9628x; 1.0763x over previous
import jax
import jax.numpy as jnp
from jax import lax
from jax.experimental import pallas as pl
from jax.experimental.pallas import tpu as pltpu
from jax.experimental.pallas import tpu_sc as plsc

H = 384
W = 384
C = 96
CP = 128
HW = H * W
NW = 32
ROWS_PER_W = H // NW
CHUNK = 96
CPR = W // CHUNK
GROUPS = CHUNK // 16


def _sc_body(table, dx_hbm, dy_hbm, out_hbm,
             dxv, dyv, idx4, w4, rowsf, outbuf,
             gsemA, gsemB, osemA, osemB):
    wid = lax.axis_index("s") * 2 + lax.axis_index("c")
    row0 = wid * ROWS_PER_W
    src0 = (H - ROWS_PER_W) - row0

    def stage_row(i, rslot):
        off = (src0 + (ROWS_PER_W - 1) - i) * W
        pltpu.sync_copy(dx_hbm.at[pl.ds(off, W)], dxv.at[rslot])
        pltpu.sync_copy(dy_hbm.at[pl.ds(off, W)], dyv.at[rslot])

    stage_row(0, 0)

    iota = lax.iota(jnp.int32, 16)

    def floor_i32(v):
        t = v.astype(jnp.int32)
        return jnp.where(v < t.astype(jnp.float32), t - 1, t)

    def compute_idx(i, cb, slot, rslot):
        rp = (H - 1) - (row0 + i)
        rp_f = jnp.full((16,), rp, dtype=jnp.int32).astype(jnp.float32)
        colbase0 = cb * CHUNK

        def idx_group(g, c2):
            dxg = dxv[rslot, pl.ds(colbase0 + g * 16, 16)]
            dyg = dyv[rslot, pl.ds(colbase0 + g * 16, 16)]
            colf = (iota + (colbase0 + g * 16)).astype(jnp.float32)
            x = (dxg + colf) + 1.0
            y = (dyg + rp_f) + 1.0
            fx = floor_i32(x)
            fy = floor_i32(y)
            px0 = jnp.clip(fx, 0, W + 1)
            px1 = jnp.clip(fx + 1, 0, W + 1)
            py0 = jnp.clip(fy, 0, H + 1)
            py1 = jnp.clip(fy + 1, 0, H + 1)
            dxw = px1.astype(jnp.float32) - x
            dyw = py1.astype(jnp.float32) - y
            wa = dxw * dyw
            wb = dxw * (1.0 - dyw)
            wc = (1.0 - dxw) * dyw
            wd = (1.0 - dxw) * (1.0 - dyw)
            okx0 = (px0 >= 1) & (px0 <= W)
            okx1 = (px1 >= 1) & (px1 <= W)
            oky0 = (py0 >= 1) & (py0 <= H)
            oky1 = (py1 >= 1) & (py1 <= H)
            cx0 = jnp.clip(px0 - 1, 0, W - 1)
            cx1 = jnp.clip(px1 - 1, 0, W - 1)
            cy0 = jnp.clip(py0 - 1, 0, H - 1) * W
            cy1 = jnp.clip(py1 - 1, 0, H - 1) * W
            sl = pl.ds(g * 16, 16)
            idx4[slot, 0, sl] = cy0 + cx0
            idx4[slot, 1, sl] = cy0 + cx1
            idx4[slot, 2, sl] = cy1 + cx0
            idx4[slot, 3, sl] = cy1 + cx1
            zero = jnp.zeros((16,), jnp.float32)
            w4[slot, 0, sl] = jnp.where(okx0 & oky0, wa, zero)
            w4[slot, 1, sl] = jnp.where(okx1 & oky0, wb, zero)
            w4[slot, 2, sl] = jnp.where(okx0 & oky1, wc, zero)
            w4[slot, 3, sl] = jnp.where(okx1 & oky1, wd, zero)
            return c2

        lax.fori_loop(0, GROUPS, idx_group, 0)

    def fire_gathers(slot, gsem):
        for t in range(4):
            pltpu.async_copy(table.at[idx4.at[slot, t]], rowsf.at[slot, t],
                             gsem)

    def wait_gathers(slot, gsem):
        for t in range(4):
            pltpu.make_async_copy(table.at[idx4.at[slot, t]],
                                  rowsf.at[slot, t], gsem).wait()

    def combine(slot):
        def g_body(g, c2):
            sl = pl.ds(g * 16, 16)
            wa_v = w4[slot, 0, sl]
            wb_v = w4[slot, 1, sl]
            wc_v = w4[slot, 2, sl]
            wd_v = w4[slot, 3, sl]
            for p16 in range(16):
                p = g * 16 + p16
                wa_s = wa_v[p16]
                wb_s = wb_v[p16]
                wc_s = wc_v[p16]
                wd_s = wd_v[p16]
                for k in range(C // 16):
                    slk = pl.ds(k * 16, 16)
                    acc = wa_s * rowsf[slot, 0, p, slk] \
                        + wb_s * rowsf[slot, 1, p, slk] \
                        + wc_s * rowsf[slot, 2, p, slk] \
                        + wd_s * rowsf[slot, 3, p, slk]
                    outbuf[0, p, slk] = acc
            return c2

        lax.fori_loop(0, GROUPS, g_body, 0)

    def out_slice(i, cb):
        pixbase = (row0 + i) * W + cb * CHUNK
        return out_hbm.at[pl.ds(pixbase, CHUNK)]

    def fire_out(i, cb, osem):
        pltpu.async_copy(outbuf.at[0], out_slice(i, cb), osem)

    def wait_out(i, cb, osem):
        pltpu.make_async_copy(outbuf.at[0], out_slice(i, cb), osem).wait()

    gsems = (gsemA, gsemB)
    osems = (osemA, osemB)

    compute_idx(0, 0, 0, 0)
    fire_gathers(0, gsemA)

    def row_body(i, carry):
        rslot = lax.bitwise_and(i, 1)
        nrslot = lax.bitwise_and(i + 1, 1)
        for cb in range(CPR):
            slot = cb & 1
            nslot = 1 - slot
            if cb == 0:
                @pl.when(i < ROWS_PER_W - 1)
                def _():
                    stage_row(i + 1, nrslot)
            if cb < CPR - 1:
                compute_idx(i, cb + 1, nslot, rslot)
                fire_gathers(nslot, gsems[nslot])
            else:
                @pl.when(i < ROWS_PER_W - 1)
                def _():
                    compute_idx(i + 1, 0, nslot, nrslot)
                    fire_gathers(nslot, gsems[nslot])

            wait_gathers(slot, gsems[slot])

            if cb >= 1:
                wait_out(i, cb - 1, osemA)
            else:
                @pl.when(i > 0)
                def _():
                    wait_out(i - 1, CPR - 1, osemA)

            combine(slot)
            fire_out(i, cb, osemA)
        return carry

    lax.fori_loop(0, ROWS_PER_W, row_body, 0)
    wait_out(ROWS_PER_W - 1, CPR - 1, osemA)


@jax.jit
def _sc_sample(table, dx, dy):
    mesh = plsc.VectorSubcoreMesh(
        core_axis_name="c", subcore_axis_name="s", num_cores=2,
        num_subcores=16)
    return pl.kernel(
        _sc_body,
        out_type=jax.ShapeDtypeStruct((HW, C), jnp.float32),
        mesh=mesh,
        compiler_params=pltpu.CompilerParams(use_tc_tiling_on_sc=True),
        scratch_types=[
            pltpu.VMEM((2, W), jnp.float32),
            pltpu.VMEM((2, W), jnp.float32),
            pltpu.VMEM((2, 4, CHUNK), jnp.int32),
            pltpu.VMEM((2, 4, CHUNK), jnp.float32),
            pltpu.VMEM((2, 4, CHUNK, CP), jnp.float32),
            pltpu.VMEM((1, CHUNK, C), jnp.float32),
            pltpu.SemaphoreType.DMA,
            pltpu.SemaphoreType.DMA,
            pltpu.SemaphoreType.DMA,
            pltpu.SemaphoreType.DMA,
        ],
    )(table, dx, dy)


def kernel(I, dx_t, dy_t):
    t = jnp.transpose(I[0].reshape(C, HW))
    table = jnp.pad(t, ((0, 0), (0, CP - C)))
    out_flat = _sc_sample(table, dx_t.reshape(HW), dy_t.reshape(HW))
    return jnp.transpose(out_flat.reshape(1, H, W, C), (0, 3, 1, 2))
